# baseline (speedup 1.0000x reference)
import functools

import jax
import jax.numpy as jnp
from jax.experimental import pallas as pl

B, N_L, N_P = 8, 32, 512
C_IN = 128
HID = 256
NG = 10
MAX_ATOMS = 14
BLOCK_I = 8


def _elu(x):
    return jnp.where(x > 0, x, jnp.exp(jnp.minimum(x, 0.0)) - 1.0)


def _proj_kernel(hl_ref, hp_ref, w1_ref, b1_ref, gamma_ref, beta_ref,
                 mean_ref, var_ref, a_ref, p_ref, t_ref):
    s = gamma_ref[:] * jax.lax.rsqrt(var_ref[:] + 1e-5)
    w = w1_ref[:] * s
    hl = hl_ref[:].reshape(B * N_L, C_IN)
    hp = hp_ref[:].reshape(B * N_P, C_IN)
    a_ref[:] = jnp.dot(hl, w[:C_IN, :], preferred_element_type=jnp.float32)
    p_ref[:] = jnp.dot(hp, w[C_IN:, :], preferred_element_type=jnp.float32)
    t_ref[:] = (b1_ref[:] - mean_ref[:]) * s + beta_ref[:]


def _pair_kernel(a_ref, p_ref, t_ref, m_ref,
                 wpi_ref, wsig_ref, wmu_ref,
                 bpi_ref, bsig_ref, bmu_ref,
                 pi_ref, sig_ref, mu_ref):
    a = a_ref[:]
    p = p_ref[0]
    m = m_ref[0]
    x = a[:, None, :] + p[None, :, :]
    x = x * m[:, :, None] + t_ref[:]
    h = _elu(x.reshape(BLOCK_I * N_P, HID))

    ypi = jnp.dot(h, wpi_ref[:], preferred_element_type=jnp.float32) + bpi_ref[:]
    zpi = jnp.exp(ypi - jnp.max(ypi, axis=-1, keepdims=True))
    pi_ref[:] = zpi / jnp.sum(zpi, axis=-1, keepdims=True)

    ys = jnp.dot(h, wsig_ref[:], preferred_element_type=jnp.float32) + bsig_ref[:]
    sig_ref[:] = _elu(ys) + 1.1

    ym = jnp.dot(h, wmu_ref[:], preferred_element_type=jnp.float32) + bmu_ref[:]
    mu_ref[:] = _elu(ym) + 1.0


@functools.partial(jax.jit, static_argnames=("interpret",))
def _run(h_l_x, l_mask, h_p_x, p_mask, W1, b1, gamma, beta,
         running_mean, running_var, W_pi, b_pi, W_sigma, b_sigma, W_mu, b_mu,
         interpret=False):
    f32 = jnp.float32
    row2 = lambda v: v.reshape(1, -1).astype(f32)

    a, p, t = pl.pallas_call(
        _proj_kernel,
        out_shape=(
            jax.ShapeDtypeStruct((B * N_L, HID), f32),
            jax.ShapeDtypeStruct((B * N_P, HID), f32),
            jax.ShapeDtypeStruct((1, HID), f32),
        ),
        interpret=interpret,
    )(h_l_x, h_p_x, W1, row2(b1), row2(gamma), row2(beta),
      row2(running_mean), row2(running_var))

    pair_mask = (l_mask[:, :, None] & p_mask[:, None, :]).astype(f32)
    p3 = p.reshape(B, N_P, HID)

    n_ib = N_L // BLOCK_I
    rows_blk = BLOCK_I * N_P
    grid = (B, n_ib)

    out_sds = jax.ShapeDtypeStruct((B * N_L * N_P, NG * MAX_ATOMS), f32)
    out_spec = pl.BlockSpec((rows_blk, NG * MAX_ATOMS),
                            lambda b, i: (b * n_ib + i, 0))
    full = lambda shape: pl.BlockSpec(shape, lambda b, i: (0,) * len(shape))

    pi, sigma, mu = pl.pallas_call(
        _pair_kernel,
        grid=grid,
        in_specs=[
            pl.BlockSpec((BLOCK_I, HID), lambda b, i: (b * n_ib + i, 0)),
            pl.BlockSpec((1, N_P, HID), lambda b, i: (b, 0, 0)),
            full((1, HID)),
            pl.BlockSpec((1, BLOCK_I, N_P), lambda b, i: (b, i, 0)),
            full((HID, NG * MAX_ATOMS)),
            full((HID, NG * MAX_ATOMS)),
            full((HID, NG * MAX_ATOMS)),
            full((1, NG * MAX_ATOMS)),
            full((1, NG * MAX_ATOMS)),
            full((1, NG * MAX_ATOMS)),
        ],
        out_specs=(out_spec, out_spec, out_spec),
        out_shape=(out_sds, out_sds, out_sds),
        interpret=interpret,
    )(a, p3, t, pair_mask, W_pi, W_sigma, W_mu,
      row2(b_pi), row2(b_sigma), row2(b_mu))
    shape3 = (B * N_L * N_P, NG, MAX_ATOMS)
    return pi.reshape(shape3), sigma.reshape(shape3), mu.reshape(shape3)


def kernel(h_l_x, l_mask, h_p_x, p_mask, W1, b1, gamma, beta, running_mean,
           running_var, W_pi, b_pi, W_sigma, b_sigma, W_mu, b_mu):
    return _run(h_l_x, l_mask, h_p_x, p_mask, W1, b1, gamma, beta,
                running_mean, running_var, W_pi, b_pi, W_sigma, b_sigma,
                W_mu, b_mu)
